# Initial kernel scaffold; baseline (speedup 1.0000x reference)
import jax
import jax.numpy as jnp
from jax.experimental import pallas as pl


def kernel(node_pairs, node_embeds, node_types, neighbor_data, W_beta_w, W_beta_b):
    raise NotImplementedError("write your pallas kernel here")


# speedup vs baseline: 4.3712x; 4.3712x over previous
import functools

import jax
import jax.numpy as jnp
from jax import lax
from jax.experimental import pallas as pl
from jax.experimental.pallas import tpu as pltpu
from jax.experimental.pallas import tpu_sc as plsc

N, D, R, K, P = 10000, 256, 4, 8, 4096
L = 16
NW = 32
ROWS_W = 2 * P // NW
RK = R * K
NODES_C = 4
CHUNK_ROWS = NODES_C * RK
CB = NODES_C * R
NCHUNK = ROWS_W // NODES_C


def _sc_body(pairs_hbm, nbr_hbm, emb_hbm, out_hbm,
             pair_v, nbr_v, rowsA, rowsB, accA, accB,
             semA, semB, sem_nbr, osemA, osemB):
    wid = lax.axis_index("s") * 2 + lax.axis_index("c")
    row_base = wid * ROWS_W

    pltpu.sync_copy(pairs_hbm.at[pl.ds(row_base, ROWS_W)], pair_v)
    pltpu.make_async_copy(nbr_hbm.at[pair_v], nbr_v, sem_nbr).start()
    pltpu.make_async_copy(nbr_hbm.at[pair_v], nbr_v, sem_nbr).wait()

    def gathers(c, buf, sem):
        cps = []
        for s in range(NODES_C):
            idx = nbr_v.at[c * NODES_C + s]
            cps.append(pltpu.make_async_copy(
                emb_hbm.at[idx], buf.at[pl.ds(s * RK, RK)], sem))
        return cps

    def fire(c, buf, sem):
        for cp in gathers(c, buf, sem):
            cp.start()

    def drain(c, buf, sem):
        for cp in gathers(c, buf, sem):
            cp.wait()

    def out_copy(c, acc, sem):
        return pltpu.make_async_copy(
            acc, out_hbm.at[pl.ds((row_base + c * NODES_C) * R, CB)], sem)

    def accumulate(buf, acc):
        def bucket(b, _):
            for cc in range(D // L):
                s = buf[b * K, pl.ds(cc * L, L)]
                for rr in range(1, K):
                    s = s + buf[b * K + rr, pl.ds(cc * L, L)]
                acc[b, pl.ds(cc * L, L)] = s
            return _
        lax.fori_loop(0, CB, bucket, None)

    fire(0, rowsA, semA)
    fire(1, rowsB, semB)

    def step(i, _):
        c0 = 2 * i
        c1 = 2 * i + 1
        drain(c0, rowsA, semA)

        @pl.when(i > 0)
        def _():
            out_copy(c0 - 2, accA, osemA).wait()
        accumulate(rowsA, accA)
        out_copy(c0, accA, osemA).start()

        @pl.when(c0 + 2 < NCHUNK)
        def _():
            fire(c0 + 2, rowsA, semA)

        drain(c1, rowsB, semB)

        @pl.when(i > 0)
        def _():
            out_copy(c1 - 2, accB, osemB).wait()
        accumulate(rowsB, accB)
        out_copy(c1, accB, osemB).start()

        @pl.when(c1 + 2 < NCHUNK)
        def _():
            fire(c1 + 2, rowsB, semB)
        return _

    lax.fori_loop(0, NCHUNK // 2, step, None)
    out_copy(NCHUNK - 2, accA, osemA).wait()
    out_copy(NCHUNK - 1, accB, osemB).wait()


@jax.jit
def _sc_gather_sum(pair_nodes, nbr_flat, node_embeds):
    mesh = plsc.VectorSubcoreMesh(core_axis_name="c", subcore_axis_name="s")
    return pl.kernel(
        _sc_body,
        out_type=jax.ShapeDtypeStruct((2 * P * R, D), jnp.float32),
        mesh=mesh,
        compiler_params=pltpu.CompilerParams(use_tc_tiling_on_sc=False),
        scratch_types=[
            pltpu.VMEM((ROWS_W,), jnp.int32),
            pltpu.VMEM((ROWS_W, RK), jnp.int32),
            pltpu.VMEM((CHUNK_ROWS, D), jnp.float32),
            pltpu.VMEM((CHUNK_ROWS, D), jnp.float32),
            pltpu.VMEM((CB, D), jnp.float32),
            pltpu.VMEM((CB, D), jnp.float32),
            pltpu.SemaphoreType.DMA,
            pltpu.SemaphoreType.DMA,
            pltpu.SemaphoreType.DMA,
            pltpu.SemaphoreType.DMA,
            pltpu.SemaphoreType.DMA,
        ],
    )(pair_nodes, nbr_flat, node_embeds)


def _tc_body(g_ref, w_ref, b_ref, o_ref):
    x = g_ref[:, 0, :] + g_ref[:, 1, :]
    acc = jnp.dot(x, w_ref[...], preferred_element_type=jnp.float32)
    o_ref[...] = jax.nn.sigmoid(acc + b_ref[...])


def _tc_matmul(g3, w_cat, bias):
    blk = 512
    return pl.pallas_call(
        _tc_body,
        grid=(P // blk,),
        in_specs=[
            pl.BlockSpec((blk, 2, R * D), lambda i: (i, 0, 0)),
            pl.BlockSpec((R * D, D), lambda i: (0, 0)),
            pl.BlockSpec((1, D), lambda i: (0, 0)),
        ],
        out_specs=pl.BlockSpec((blk, D), lambda i: (i, 0)),
        out_shape=jax.ShapeDtypeStruct((P, D), jnp.float32),
    )(g3, w_cat, bias)


def kernel(node_pairs, node_embeds, node_types, neighbor_data, W_beta_w, W_beta_b):
    del node_types
    pair_nodes = node_pairs.reshape(-1).astype(jnp.int32)
    nbr_flat = neighbor_data.reshape(N, RK).astype(jnp.int32)
    g = _sc_gather_sum(pair_nodes, nbr_flat, node_embeds)
    w_cat = (jnp.transpose(W_beta_w, (0, 2, 1)).reshape(R * D, D)
             * (1.0 / (2 * K * R))).astype(jnp.float32)
    bias = jnp.mean(W_beta_b, axis=0, keepdims=True)
    return _tc_matmul(g.reshape(P, 2, R * D), w_cat, bias)


# speedup vs baseline: 5.1357x; 1.1749x over previous
import functools

import jax
import jax.numpy as jnp
from jax import lax
from jax.experimental import pallas as pl
from jax.experimental.pallas import tpu as pltpu
from jax.experimental.pallas import tpu_sc as plsc

N, D, R, K, P = 10000, 256, 4, 8, 4096
L = 16
NW = 32
ROWS_W = 2 * P // NW
RK = R * K
NODES_C = 4
CHUNK_ROWS = NODES_C * RK
CB = NODES_C * R
NCHUNK = ROWS_W // NODES_C


def _sc_body(pairs_hbm, nbr_hbm, emb_hbm, out_hbm,
             pair_v, nbr_v, rowsA, rowsB, accA, accB,
             semA, semB, sem_nbr, osemA, osemB):
    wid = lax.axis_index("s") * 2 + lax.axis_index("c")
    row_base = wid * ROWS_W

    pltpu.sync_copy(pairs_hbm.at[pl.ds(row_base, ROWS_W)], pair_v)
    pltpu.make_async_copy(nbr_hbm.at[pair_v], nbr_v, sem_nbr).start()
    pltpu.make_async_copy(nbr_hbm.at[pair_v], nbr_v, sem_nbr).wait()

    def gathers(c, buf, sem):
        cps = []
        for s in range(NODES_C):
            idx = nbr_v.at[c * NODES_C + s]
            cps.append(pltpu.make_async_copy(
                emb_hbm.at[idx], buf.at[pl.ds(s * RK, RK)], sem))
        return cps

    def fire(c, buf, sem):
        for cp in gathers(c, buf, sem):
            cp.start()

    def drain(c, buf, sem):
        for cp in gathers(c, buf, sem):
            cp.wait()

    def out_copy(c, acc, sem):
        return pltpu.make_async_copy(
            acc, out_hbm.at[pl.ds((row_base + c * NODES_C) * R, CB)], sem)

    def accumulate(buf, acc):
        L2 = 2 * L
        def bucket(b, _):
            for cc in range(D // L2):
                s = buf[b * K, pl.ds(cc * L2, L2)]
                for rr in range(1, K):
                    s = s + buf[b * K + rr, pl.ds(cc * L2, L2)]

                acc[b, pl.ds(cc * L2, L2)] = s
            return _
        lax.fori_loop(0, CB, bucket, None)

    fire(0, rowsA, semA)
    fire(1, rowsB, semB)

    def step(i, _):
        c0 = 2 * i
        c1 = 2 * i + 1
        drain(c0, rowsA, semA)

        @pl.when(i > 0)
        def _():
            out_copy(c0 - 2, accA, osemA).wait()
        accumulate(rowsA, accA)
        out_copy(c0, accA, osemA).start()

        @pl.when(c0 + 2 < NCHUNK)
        def _():
            fire(c0 + 2, rowsA, semA)

        drain(c1, rowsB, semB)

        @pl.when(i > 0)
        def _():
            out_copy(c1 - 2, accB, osemB).wait()
        accumulate(rowsB, accB)
        out_copy(c1, accB, osemB).start()

        @pl.when(c1 + 2 < NCHUNK)
        def _():
            fire(c1 + 2, rowsB, semB)
        return _

    lax.fori_loop(0, NCHUNK // 2, step, None)
    out_copy(NCHUNK - 2, accA, osemA).wait()
    out_copy(NCHUNK - 1, accB, osemB).wait()


@jax.jit
def _sc_gather_sum(pair_nodes, nbr_flat, node_embeds):
    mesh = plsc.VectorSubcoreMesh(core_axis_name="c", subcore_axis_name="s")
    return pl.kernel(
        _sc_body,
        out_type=jax.ShapeDtypeStruct((2 * P * R, D), jnp.bfloat16),
        mesh=mesh,
        compiler_params=pltpu.CompilerParams(use_tc_tiling_on_sc=False),
        scratch_types=[
            pltpu.VMEM((ROWS_W,), jnp.int32),
            pltpu.VMEM((ROWS_W, RK), jnp.int32),
            pltpu.VMEM((CHUNK_ROWS, D), jnp.bfloat16),
            pltpu.VMEM((CHUNK_ROWS, D), jnp.bfloat16),
            pltpu.VMEM((CB, D), jnp.bfloat16),
            pltpu.VMEM((CB, D), jnp.bfloat16),
            pltpu.SemaphoreType.DMA,
            pltpu.SemaphoreType.DMA,
            pltpu.SemaphoreType.DMA,
            pltpu.SemaphoreType.DMA,
            pltpu.SemaphoreType.DMA,
        ],
    )(pair_nodes, nbr_flat, node_embeds)


def _tc_body(g_ref, w_ref, b_ref, o_ref):
    x = g_ref[:, 0, :] + g_ref[:, 1, :]
    acc = jnp.dot(x, w_ref[...], preferred_element_type=jnp.float32)
    o_ref[...] = jax.nn.sigmoid(acc + b_ref[...])


def _tc_matmul(g3, w_cat, bias):
    blk = 512
    return pl.pallas_call(
        _tc_body,
        grid=(P // blk,),
        in_specs=[
            pl.BlockSpec((blk, 2, R * D), lambda i: (i, 0, 0)),
            pl.BlockSpec((R * D, D), lambda i: (0, 0)),
            pl.BlockSpec((1, D), lambda i: (0, 0)),
        ],
        out_specs=pl.BlockSpec((blk, D), lambda i: (i, 0)),
        out_shape=jax.ShapeDtypeStruct((P, D), jnp.float32),
    )(g3, w_cat, bias)


def kernel(node_pairs, node_embeds, node_types, neighbor_data, W_beta_w, W_beta_b):
    del node_types
    pair_nodes = node_pairs.reshape(-1).astype(jnp.int32)
    nbr_flat = neighbor_data.reshape(N, RK).astype(jnp.int32)
    g = _sc_gather_sum(pair_nodes, nbr_flat,
                       node_embeds.astype(jnp.bfloat16))
    w_cat = (jnp.transpose(W_beta_w, (0, 2, 1)).reshape(R * D, D)
             * (1.0 / (2 * K * R))).astype(jnp.bfloat16)
    bias = jnp.mean(W_beta_b, axis=0, keepdims=True)
    return _tc_matmul(g.reshape(P, 2, R * D), w_cat, bias)
